# baseline (speedup 1.0000x reference)
import jax
import jax.numpy as jnp
from jax.experimental import pallas as pl


def _score_body(ehre, ehim, etre, etim, rre, rim, out):
    a = ehre[...]
    b = ehim[...]
    c = etre[...]
    d = etim[...]
    p = rre[...]
    q = rim[...]
    out[...] = jnp.sum(p * (a * c + b * d) + q * (a * d - b * c), axis=1)


def kernel(hs, rs, ts, ent_re, ent_im, rel_re, rel_im):
    e_re_h = jnp.take(ent_re, hs, axis=0)
    e_im_h = jnp.take(ent_im, hs, axis=0)
    e_re_t = jnp.take(ent_re, ts, axis=0)
    e_im_t = jnp.take(ent_im, ts, axis=0)
    r_re = jnp.take(rel_re, rs, axis=0)
    r_im = jnp.take(rel_im, rs, axis=0)
    batch = hs.shape[0]
    return pl.pallas_call(
        _score_body,
        out_shape=jax.ShapeDtypeStruct((batch,), jnp.float32),
    )(e_re_h, e_im_h, e_re_t, e_im_t, r_re, r_im)
